# baseline (speedup 1.0000x reference)
import functools

import jax
import jax.numpy as jnp
from jax import lax
from jax.experimental import pallas as pl
from jax.experimental.pallas import tpu as pltpu
from jax.experimental.pallas import tpu_sc as plsc

_H = 512
_W = 512
_BOX = 40
_B = 8
_K = 5
_NB = _B * _K
_GRP = 128
_GPB = 2
_ROWG = _BOX * _GPB


def _box_starts(centroids):
    y = centroids[..., 0].astype(jnp.int32)
    x = centroids[..., 1].astype(jnp.int32)
    half = _BOX // 2
    start_x = jnp.maximum(x - half, 0)
    start_y = jnp.maximum(y - half, 0)
    end_x = jnp.minimum(x + half, _W)
    end_y = jnp.minimum(y + half, _H)
    new_w = end_x - start_x
    w_odd = (new_w % 2) != 0
    end_x = jnp.where(w_odd & (new_w < _BOX) & (start_x == 0), end_x - 1, end_x)
    start_x = jnp.where(w_odd & (new_w < _BOX) & (end_x == _W), start_x + 1, start_x)
    new_h = end_y - start_y
    h_odd = (new_h % 2) != 0
    end_y = jnp.where(h_odd & (new_h < _BOX) & (start_y == 0), end_y - 1, end_y)
    start_y = jnp.where(h_odd & (new_h < _BOX) & (end_y == _H), start_y + 1, start_y)
    sx = jnp.clip(start_x, 0, _W - _BOX)
    sy = jnp.clip(start_y, 0, _H - _BOX)
    return sy, sx


_HB = _BOX // 2
_NT = _NB * 2
_PADR = 24


def _sc_region_kernel(ptbl, gtbl, meta_h, out_h,
                      mv0, mv1, pb0, gb0, pb1, gb1, stage,
                      sp0, sg0, sp1, sg1):
    info = plsc.get_sparse_core_info()
    nc = info.num_cores
    wid = lax.axis_index("s") * nc + lax.axis_index("c")
    lane = lax.broadcasted_iota(jnp.int32, (16,), 0)

    def start(t, mv, pb, gb, semp, semg):
        pltpu.sync_copy(meta_h.at[t], mv)
        cp = pltpu.async_copy(ptbl.at[mv.at[pl.ds(0, _PADR)]], pb, semp)
        cg = pltpu.async_copy(gtbl.at[mv.at[pl.ds(0, _PADR)]], gb, semg)
        return cp, cg

    def compute(t, mv, pb, gb, cp, cg):
        b = t // 2
        h = t - b * 2
        i = b // _K
        k = b - i * _K
        cp.wait()
        cg.wait()
        dv = mv[pl.ds(32, 16)]
        dvp = dv + _BOX
        masks = []
        for l in range(_W // 16):
            off = lane + (16 * l)
            masks.append(jnp.where((off >= dv) & (off < dvp), 1.0, 0.0))

        def body(j, accs):
            accs = list(accs)
            for rr in range(2):
                for l in range(_W // 16):
                    vp = pb[2 * j + rr, pl.ds(16 * l, 16)]
                    vg = gb[2 * j + rr, pl.ds(16 * l, 16)]
                    pm = vp * masks[l]
                    w = l % 4
                    accs[w] = accs[w] + pm
                    accs[4 + w] = accs[4 + w] + vg * masks[l]
                    accs[8 + w] = accs[8 + w] + pm * vg
            return tuple(accs)

        z = jnp.zeros((16,), jnp.float32)
        accs = lax.fori_loop(0, _HB // 2, body, (z,) * 12)
        stage[pl.ds(0, 16)] = accs[0] + accs[1] + accs[2] + accs[3]
        stage[pl.ds(16, 16)] = accs[4] + accs[5] + accs[6] + accs[7]
        stage[pl.ds(32, 16)] = accs[8] + accs[9] + accs[10] + accs[11]
        pltpu.sync_copy(stage, out_h.at[h, i, k])

    c0 = start(wid, mv0, pb0, gb0, sp0, sg0)
    c1 = start(wid + 32, mv1, pb1, gb1, sp1, sg1)
    compute(wid, mv0, pb0, gb0, *c0)

    @pl.when(wid < _NT - 64)
    def _():
        c2 = start(wid + 64, mv0, pb0, gb0, sp0, sg0)
        compute(wid + 32, mv1, pb1, gb1, *c1)
        compute(wid + 64, mv0, pb0, gb0, *c2)

    @pl.when(wid >= _NT - 64)
    def _():
        compute(wid + 32, mv1, pb1, gb1, *c1)


_TCH = 16
_TROWS = _B * _H // _TCH
_TNBUF = 4


def _totals_kernel(p_hbm, g_hbm, out_ref, pbufs, gbufs, psems, gsems):
    def copies(c):
        slot = c % _TNBUF
        cp = pltpu.make_async_copy(
            p_hbm.at[pl.ds(c * _TROWS, _TROWS), :], pbufs.at[slot], psems.at[slot])
        cg = pltpu.make_async_copy(
            g_hbm.at[pl.ds(c * _TROWS, _TROWS), :], gbufs.at[slot], gsems.at[slot])
        return cp, cg

    for c in range(_TNBUF):
        cp, cg = copies(c)
        cp.start()
        cg.start()
    tp = [jnp.float32(0.0)] * _B
    tg = [jnp.float32(0.0)] * _B
    tpg = [jnp.float32(0.0)] * _B
    for c in range(_TCH):
        cp, cg = copies(c)
        cp.wait()
        cg.wait()
        nxt = c + _TNBUF
        if nxt < _TCH:
            cp2, cg2 = copies(nxt)
            cp2.start()
            cg2.start()
        slot = c % _TNBUF
        p = pbufs[slot]
        g = gbufs[slot]
        i = c // (_TCH // _B)
        tp[i] = tp[i] + jnp.sum(p)
        tg[i] = tg[i] + jnp.sum(g)
        tpg[i] = tpg[i] + jnp.sum(p * g)
    li = lax.broadcasted_iota(jnp.int32, (_B, 128), 1)
    ri = lax.broadcasted_iota(jnp.int32, (_B, 128), 0)
    acc = jnp.zeros((_B, 128), jnp.float32)
    for i in range(_B):
        row = jnp.where(li == 0, tp[i],
                        jnp.where(li == 1, tg[i],
                                  jnp.where(li == 2, tpg[i], 0.0)))
        acc = acc + jnp.where(ri == i, row, 0.0)
    out_ref[...] = acc


def _combine_kernel(tot_ref, sc_ref, out_ref):
    sp = (jnp.sum(sc_ref[0, :, :, 0:16], axis=2)
          + jnp.sum(sc_ref[1, :, :, 0:16], axis=2))
    sg = (jnp.sum(sc_ref[0, :, :, 16:32], axis=2)
          + jnp.sum(sc_ref[1, :, :, 16:32], axis=2))
    spg = (jnp.sum(sc_ref[0, :, :, 32:48], axis=2)
           + jnp.sum(sc_ref[1, :, :, 32:48], axis=2))
    t = tot_ref[...]
    tp = t[:, 0:1]
    tg = t[:, 1:2]
    tpg = t[:, 2:3]
    iou = (spg + 1.0) / (sp + sg - spg + 1.0)
    alpha = (1.0 + jnp.cos(jnp.pi * iou)) * 0.5
    region = jnp.sum(alpha * iou, axis=1, keepdims=True)
    bp = jnp.sum(sp, axis=1, keepdims=True)
    bg = jnp.sum(sg, axis=1, keepdims=True)
    bpg = jnp.sum(spg, axis=1, keepdims=True)
    cp = tp - bp
    cg = tg - bg
    cpg = tpg - bpg
    soft = (cpg + 1.0) / (cp + cg - cpg + 1.0)
    loss = (soft + region) / jnp.float32(_K)
    out_ref[0, 0] = 1.0 - jnp.sum(loss) / jnp.float32(_B)


@jax.jit
def kernel(preds, gt_masks, centroids):
    sy, sx = _box_starts(centroids)

    bi = jnp.arange(_B, dtype=jnp.int32)[:, None, None, None]
    hh = jnp.arange(2, dtype=jnp.int32)[None, None, :, None]
    cc = jnp.arange(48, dtype=jnp.int32)[None, None, None, :]
    r = jnp.minimum(cc, _HB - 1)
    rowid = bi * _H + sy[:, :, None, None] + hh * _HB + r
    meta = jnp.where(cc < _PADR, rowid,
                     jnp.where(cc >= 32, sx[:, :, None, None], 0)
                     ).reshape(_NT, 48)

    ptbl = preds.reshape(_B * _H, _W)
    gtbl = gt_masks.reshape(_B * _H, _W)

    sc_sums = pl.kernel(
        _sc_region_kernel,
        out_type=jax.ShapeDtypeStruct((2, _B, _K, 48), jnp.float32),
        mesh=plsc.VectorSubcoreMesh(core_axis_name="c", subcore_axis_name="s"),
        scratch_types=[
            pltpu.VMEM((48,), jnp.int32),
            pltpu.VMEM((48,), jnp.int32),
            pltpu.VMEM((_PADR, _W), jnp.float32),
            pltpu.VMEM((_PADR, _W), jnp.float32),
            pltpu.VMEM((_PADR, _W), jnp.float32),
            pltpu.VMEM((_PADR, _W), jnp.float32),
            pltpu.VMEM((48,), jnp.float32),
            pltpu.SemaphoreType.DMA,
            pltpu.SemaphoreType.DMA,
            pltpu.SemaphoreType.DMA,
            pltpu.SemaphoreType.DMA,
        ],
    )(ptbl, gtbl, meta)

    totals = pl.pallas_call(
        _totals_kernel,
        in_specs=[
            pl.BlockSpec(memory_space=pl.ANY),
            pl.BlockSpec(memory_space=pl.ANY),
        ],
        out_specs=pl.BlockSpec(memory_space=pltpu.VMEM),
        out_shape=jax.ShapeDtypeStruct((_B, 128), jnp.float32),
        scratch_shapes=[
            pltpu.VMEM((_TNBUF, _TROWS, _W), jnp.float32),
            pltpu.VMEM((_TNBUF, _TROWS, _W), jnp.float32),
            pltpu.SemaphoreType.DMA((_TNBUF,)),
            pltpu.SemaphoreType.DMA((_TNBUF,)),
        ],
    )(ptbl, gtbl)

    out = pl.pallas_call(
        _combine_kernel,
        out_specs=pl.BlockSpec(memory_space=pltpu.SMEM),
        out_shape=jax.ShapeDtypeStruct((1, 1), jnp.float32),
    )(totals, sc_sums)
    return out[0, 0]


# speedup vs baseline: 1.2389x; 1.2389x over previous
import functools

import jax
import jax.numpy as jnp
from jax import lax
from jax.experimental import pallas as pl
from jax.experimental.pallas import tpu as pltpu
from jax.experimental.pallas import tpu_sc as plsc

_H = 512
_W = 512
_BOX = 40
_B = 8
_K = 5
_NB = _B * _K
_GRP = 128
_GPB = 2
_ROWG = _BOX * _GPB


def _box_starts(centroids):
    y = centroids[..., 0].astype(jnp.int32)
    x = centroids[..., 1].astype(jnp.int32)
    half = _BOX // 2
    start_x = jnp.maximum(x - half, 0)
    start_y = jnp.maximum(y - half, 0)
    end_x = jnp.minimum(x + half, _W)
    end_y = jnp.minimum(y + half, _H)
    new_w = end_x - start_x
    w_odd = (new_w % 2) != 0
    end_x = jnp.where(w_odd & (new_w < _BOX) & (start_x == 0), end_x - 1, end_x)
    start_x = jnp.where(w_odd & (new_w < _BOX) & (end_x == _W), start_x + 1, start_x)
    new_h = end_y - start_y
    h_odd = (new_h % 2) != 0
    end_y = jnp.where(h_odd & (new_h < _BOX) & (start_y == 0), end_y - 1, end_y)
    start_y = jnp.where(h_odd & (new_h < _BOX) & (end_y == _H), start_y + 1, start_y)
    sx = jnp.clip(start_x, 0, _W - _BOX)
    sy = jnp.clip(start_y, 0, _H - _BOX)
    return sy, sx


_HB = _BOX // 2
_NT = _NB * 2
_PADR = 24


def _sc_region_kernel(ptbl, gtbl, meta_h, out_h,
                      mv0, mv1, pb0, gb0, pb1, gb1, stage,
                      sp0, sg0, sp1, sg1):
    info = plsc.get_sparse_core_info()
    nc = info.num_cores
    wid = lax.axis_index("s") * nc + lax.axis_index("c")
    lane = lax.broadcasted_iota(jnp.int32, (16,), 0)

    def start(t, mv, pb, gb, semp, semg):
        pltpu.sync_copy(meta_h.at[t], mv)
        cp = pltpu.async_copy(ptbl.at[mv.at[pl.ds(0, _PADR)]], pb, semp)
        cg = pltpu.async_copy(gtbl.at[mv.at[pl.ds(0, _PADR)]], gb, semg)
        return cp, cg

    def compute(t, mv, pb, gb, cp, cg):
        b = t // 2
        h = t - b * 2
        i = b // _K
        k = b - i * _K
        cp.wait()
        cg.wait()
        dv = mv[pl.ds(32, 16)]
        dvp = dv + _BOX
        masks = []
        for l in range(_W // 16):
            off = lane + (16 * l)
            masks.append(jnp.where((off >= dv) & (off < dvp), 1.0, 0.0))

        def body(j, accs):
            accs = list(accs)
            for l in range(_W // 16):
                vp = pb[j, pl.ds(16 * l, 16)]
                vg = gb[j, pl.ds(16 * l, 16)]
                pm = vp * masks[l]
                w = l % 4
                accs[w] = accs[w] + pm
                accs[4 + w] = accs[4 + w] + vg * masks[l]
                accs[8 + w] = accs[8 + w] + pm * vg
            return tuple(accs)

        z = jnp.zeros((16,), jnp.float32)
        accs = lax.fori_loop(0, _HB, body, (z,) * 12)
        stage[pl.ds(0, 16)] = accs[0] + accs[1] + accs[2] + accs[3]
        stage[pl.ds(16, 16)] = accs[4] + accs[5] + accs[6] + accs[7]
        stage[pl.ds(32, 16)] = accs[8] + accs[9] + accs[10] + accs[11]
        pltpu.sync_copy(stage, out_h.at[h, i, k])

    c0 = start(wid, mv0, pb0, gb0, sp0, sg0)
    c1 = start(wid + 32, mv1, pb1, gb1, sp1, sg1)
    compute(wid, mv0, pb0, gb0, *c0)

    @pl.when(wid < _NT - 64)
    def _():
        c2 = start(wid + 64, mv0, pb0, gb0, sp0, sg0)
        compute(wid + 32, mv1, pb1, gb1, *c1)
        compute(wid + 64, mv0, pb0, gb0, *c2)

    @pl.when(wid >= _NT - 64)
    def _():
        compute(wid + 32, mv1, pb1, gb1, *c1)


_TCH = 16
_TROWS = _B * _H // _TCH
_TNBUF = 4


def _totals_kernel(p_hbm, g_hbm, out_ref, pbufs, gbufs, psems, gsems):
    def copies(c):
        slot = c % _TNBUF
        cp = pltpu.make_async_copy(
            p_hbm.at[pl.ds(c * _TROWS, _TROWS), :], pbufs.at[slot], psems.at[slot])
        cg = pltpu.make_async_copy(
            g_hbm.at[pl.ds(c * _TROWS, _TROWS), :], gbufs.at[slot], gsems.at[slot])
        return cp, cg

    for c in range(_TNBUF):
        cp, cg = copies(c)
        cp.start()
        cg.start()
    tp = [jnp.float32(0.0)] * _B
    tg = [jnp.float32(0.0)] * _B
    tpg = [jnp.float32(0.0)] * _B
    for c in range(_TCH):
        cp, cg = copies(c)
        cp.wait()
        cg.wait()
        nxt = c + _TNBUF
        if nxt < _TCH:
            cp2, cg2 = copies(nxt)
            cp2.start()
            cg2.start()
        slot = c % _TNBUF
        p = pbufs[slot]
        g = gbufs[slot]
        i = c // (_TCH // _B)
        tp[i] = tp[i] + jnp.sum(p)
        tg[i] = tg[i] + jnp.sum(g)
        tpg[i] = tpg[i] + jnp.sum(p * g)
    li = lax.broadcasted_iota(jnp.int32, (_B, 128), 1)
    ri = lax.broadcasted_iota(jnp.int32, (_B, 128), 0)
    acc = jnp.zeros((_B, 128), jnp.float32)
    for i in range(_B):
        row = jnp.where(li == 0, tp[i],
                        jnp.where(li == 1, tg[i],
                                  jnp.where(li == 2, tpg[i], 0.0)))
        acc = acc + jnp.where(ri == i, row, 0.0)
    out_ref[...] = acc


def _combine_kernel(tot_ref, sc_ref, out_ref):
    sp = (jnp.sum(sc_ref[0, :, :, 0:16], axis=2)
          + jnp.sum(sc_ref[1, :, :, 0:16], axis=2))
    sg = (jnp.sum(sc_ref[0, :, :, 16:32], axis=2)
          + jnp.sum(sc_ref[1, :, :, 16:32], axis=2))
    spg = (jnp.sum(sc_ref[0, :, :, 32:48], axis=2)
           + jnp.sum(sc_ref[1, :, :, 32:48], axis=2))
    t = tot_ref[...]
    tp = t[:, 0:1]
    tg = t[:, 1:2]
    tpg = t[:, 2:3]
    iou = (spg + 1.0) / (sp + sg - spg + 1.0)
    alpha = (1.0 + jnp.cos(jnp.pi * iou)) * 0.5
    region = jnp.sum(alpha * iou, axis=1, keepdims=True)
    bp = jnp.sum(sp, axis=1, keepdims=True)
    bg = jnp.sum(sg, axis=1, keepdims=True)
    bpg = jnp.sum(spg, axis=1, keepdims=True)
    cp = tp - bp
    cg = tg - bg
    cpg = tpg - bpg
    soft = (cpg + 1.0) / (cp + cg - cpg + 1.0)
    loss = (soft + region) / jnp.float32(_K)
    out_ref[0, 0] = 1.0 - jnp.sum(loss) / jnp.float32(_B)


@jax.jit
def kernel(preds, gt_masks, centroids):
    sy, sx = _box_starts(centroids)

    bi = jnp.arange(_B, dtype=jnp.int32)[:, None, None, None]
    hh = jnp.arange(2, dtype=jnp.int32)[None, None, :, None]
    cc = jnp.arange(48, dtype=jnp.int32)[None, None, None, :]
    r = jnp.minimum(cc, _HB - 1)
    rowid = bi * _H + sy[:, :, None, None] + hh * _HB + r
    meta = jnp.where(cc < _PADR, rowid,
                     jnp.where(cc >= 32, sx[:, :, None, None], 0)
                     ).reshape(_NT, 48)

    ptbl = preds.reshape(_B * _H, _W)
    gtbl = gt_masks.reshape(_B * _H, _W)

    sc_sums = pl.kernel(
        _sc_region_kernel,
        out_type=jax.ShapeDtypeStruct((2, _B, _K, 48), jnp.float32),
        mesh=plsc.VectorSubcoreMesh(core_axis_name="c", subcore_axis_name="s"),
        scratch_types=[
            pltpu.VMEM((48,), jnp.int32),
            pltpu.VMEM((48,), jnp.int32),
            pltpu.VMEM((_PADR, _W), jnp.float32),
            pltpu.VMEM((_PADR, _W), jnp.float32),
            pltpu.VMEM((_PADR, _W), jnp.float32),
            pltpu.VMEM((_PADR, _W), jnp.float32),
            pltpu.VMEM((48,), jnp.float32),
            pltpu.SemaphoreType.DMA,
            pltpu.SemaphoreType.DMA,
            pltpu.SemaphoreType.DMA,
            pltpu.SemaphoreType.DMA,
        ],
    )(ptbl, gtbl, meta)

    totals = pl.pallas_call(
        _totals_kernel,
        in_specs=[
            pl.BlockSpec(memory_space=pl.ANY),
            pl.BlockSpec(memory_space=pl.ANY),
        ],
        out_specs=pl.BlockSpec(memory_space=pltpu.VMEM),
        out_shape=jax.ShapeDtypeStruct((_B, 128), jnp.float32),
        scratch_shapes=[
            pltpu.VMEM((_TNBUF, _TROWS, _W), jnp.float32),
            pltpu.VMEM((_TNBUF, _TROWS, _W), jnp.float32),
            pltpu.SemaphoreType.DMA((_TNBUF,)),
            pltpu.SemaphoreType.DMA((_TNBUF,)),
        ],
    )(ptbl, gtbl)

    out = pl.pallas_call(
        _combine_kernel,
        out_specs=pl.BlockSpec(memory_space=pltpu.SMEM),
        out_shape=jax.ShapeDtypeStruct((1, 1), jnp.float32),
    )(totals, sc_sums)
    return out[0, 0]


# speedup vs baseline: 1.3710x; 1.1067x over previous
import functools

import jax
import jax.numpy as jnp
from jax import lax
from jax.experimental import pallas as pl
from jax.experimental.pallas import tpu as pltpu
from jax.experimental.pallas import tpu_sc as plsc

_H = 512
_W = 512
_BOX = 40
_B = 8
_K = 5
_NB = _B * _K
_GRP = 128
_GPB = 2
_ROWG = _BOX * _GPB


def _box_starts(centroids):
    y = centroids[..., 0].astype(jnp.int32)
    x = centroids[..., 1].astype(jnp.int32)
    half = _BOX // 2
    start_x = jnp.maximum(x - half, 0)
    start_y = jnp.maximum(y - half, 0)
    end_x = jnp.minimum(x + half, _W)
    end_y = jnp.minimum(y + half, _H)
    new_w = end_x - start_x
    w_odd = (new_w % 2) != 0
    end_x = jnp.where(w_odd & (new_w < _BOX) & (start_x == 0), end_x - 1, end_x)
    start_x = jnp.where(w_odd & (new_w < _BOX) & (end_x == _W), start_x + 1, start_x)
    new_h = end_y - start_y
    h_odd = (new_h % 2) != 0
    end_y = jnp.where(h_odd & (new_h < _BOX) & (start_y == 0), end_y - 1, end_y)
    start_y = jnp.where(h_odd & (new_h < _BOX) & (end_y == _H), start_y + 1, start_y)
    sx = jnp.clip(start_x, 0, _W - _BOX)
    sy = jnp.clip(start_y, 0, _H - _BOX)
    return sy, sx


_HB = _BOX // 2
_NT = _NB * 2
_PADR = 24


def _sc_region_kernel(ptbl, meta_h, out_h,
                      mv0, mv1, pb0, pb1, stage, sp0, sp1):
    info = plsc.get_sparse_core_info()
    nc = info.num_cores
    wid = lax.axis_index("s") * nc + lax.axis_index("c")
    lane = lax.broadcasted_iota(jnp.int32, (16,), 0)

    def start(t, mv, pb, semp):
        pltpu.sync_copy(meta_h.at[t], mv)
        return pltpu.async_copy(ptbl.at[mv.at[pl.ds(0, _PADR)]], pb, semp)

    def compute(t, mv, pb, cp):
        b = t // 2
        h = t - b * 2
        i = b // _K
        k = b - i * _K
        cp.wait()
        dv = mv[pl.ds(32, 16)]
        dvp = dv + _BOX
        masks = []
        for l in range(_W // 16):
            off = lane + (16 * l)
            masks.append(jnp.where((off >= dv) & (off < dvp), 1.0, 0.0))

        def body(j, accs):
            accs = list(accs)
            for l in range(_W // 16):
                vp = pb[j, pl.ds(16 * l, 16)]
                w = l % 4
                accs[w] = accs[w] + vp * masks[l]
            return tuple(accs)

        z = jnp.zeros((16,), jnp.float32)
        accs = lax.fori_loop(0, _HB, body, (z,) * 4)
        stage[...] = accs[0] + accs[1] + accs[2] + accs[3]
        pltpu.sync_copy(stage, out_h.at[h, i, k])

    c0 = start(wid, mv0, pb0, sp0)
    c1 = start(wid + 32, mv1, pb1, sp1)
    compute(wid, mv0, pb0, c0)

    @pl.when(wid < _NT - 64)
    def _():
        c2 = start(wid + 64, mv0, pb0, sp0)
        compute(wid + 32, mv1, pb1, c1)
        compute(wid + 64, mv0, pb0, c2)

    @pl.when(wid >= _NT - 64)
    def _():
        compute(wid + 32, mv1, pb1, c1)


_TCH = 8
_TROWS = _B * _H // _TCH
_TNBUF = 3


def _totals_kernel(sy_ref, sx_ref, p_hbm, g_hbm, out_ref, pbufs, gbufs,
                   psems, gsems):
    def copies(c):
        slot = c % _TNBUF
        cp = pltpu.make_async_copy(
            p_hbm.at[pl.ds(c * _TROWS, _TROWS), :], pbufs.at[slot], psems.at[slot])
        cg = pltpu.make_async_copy(
            g_hbm.at[pl.ds(c * _TROWS, _TROWS), :], gbufs.at[slot], gsems.at[slot])
        return cp, cg

    for c in range(_TNBUF):
        cp, cg = copies(c)
        cp.start()
        cg.start()
    li = lax.broadcasted_iota(jnp.int32, (_B, 128), 1)
    ri = lax.broadcasted_iota(jnp.int32, (_B, 128), 0)
    acc = jnp.zeros((_B, 128), jnp.float32)
    for i in range(_TCH):
        cp, cg = copies(i)
        cp.wait()
        cg.wait()
        nxt = i + _TNBUF
        if nxt < _TCH:
            cp2, cg2 = copies(nxt)
            cp2.start()
            cg2.start()
        slot = i % _TNBUF
        p = pbufs[slot]
        g = gbufs[slot]
        row = jnp.where(li == 0, jnp.sum(p),
                        jnp.where(li == 1, jnp.sum(g),
                                  jnp.where(li == 2, jnp.sum(p * g), 0.0)))
        for k in range(_K):
            sy = sy_ref[i, k]
            sx = sx_ref[i, k]
            sy8 = pl.multiple_of(jnp.minimum((sy // 8) * 8, _H - 48), 8)
            sx128 = pl.multiple_of(jnp.minimum((sx // 128) * 128, _W - 256), 128)
            pb = pbufs[slot, pl.ds(sy8, 48), pl.ds(sx128, 256)]
            gb = gbufs[slot, pl.ds(sy8, 48), pl.ds(sx128, 256)]
            rows = lax.broadcasted_iota(jnp.int32, (48, 256), 0) + sy8
            cols = lax.broadcasted_iota(jnp.int32, (48, 256), 1) + sx128
            m = ((rows >= sy) & (rows < sy + _BOX)
                 & (cols >= sx) & (cols < sx + _BOX)).astype(jnp.float32)
            gm = gb * m
            row = row + jnp.where(li == 3 + k, jnp.sum(gm),
                                  jnp.where(li == 8 + k, jnp.sum(gm * pb), 0.0))
        acc = acc + jnp.where(ri == i, row, 0.0)
    out_ref[...] = acc


def _combine_kernel(tot_ref, sc_ref, out_ref):
    sp = (jnp.sum(sc_ref[0, :, :, :], axis=2)
          + jnp.sum(sc_ref[1, :, :, :], axis=2))
    t = tot_ref[...]
    sg = t[:, 3:3 + _K]
    spg = t[:, 8:8 + _K]
    tp = t[:, 0:1]
    tg = t[:, 1:2]
    tpg = t[:, 2:3]
    iou = (spg + 1.0) / (sp + sg - spg + 1.0)
    alpha = (1.0 + jnp.cos(jnp.pi * iou)) * 0.5
    region = jnp.sum(alpha * iou, axis=1, keepdims=True)
    bp = jnp.sum(sp, axis=1, keepdims=True)
    bg = jnp.sum(sg, axis=1, keepdims=True)
    bpg = jnp.sum(spg, axis=1, keepdims=True)
    cp = tp - bp
    cg = tg - bg
    cpg = tpg - bpg
    soft = (cpg + 1.0) / (cp + cg - cpg + 1.0)
    loss = (soft + region) / jnp.float32(_K)
    out_ref[0, 0] = 1.0 - jnp.sum(loss) / jnp.float32(_B)


@jax.jit
def kernel(preds, gt_masks, centroids):
    sy, sx = _box_starts(centroids)

    bi = jnp.arange(_B, dtype=jnp.int32)[:, None, None, None]
    hh = jnp.arange(2, dtype=jnp.int32)[None, None, :, None]
    cc = jnp.arange(48, dtype=jnp.int32)[None, None, None, :]
    r = jnp.minimum(cc, _HB - 1)
    rowid = bi * _H + sy[:, :, None, None] + hh * _HB + r
    meta = jnp.where(cc < _PADR, rowid,
                     jnp.where(cc >= 32, sx[:, :, None, None], 0)
                     ).reshape(_NT, 48)

    ptbl = preds.reshape(_B * _H, _W)
    gtbl = gt_masks.reshape(_B * _H, _W)

    sc_sums = pl.kernel(
        _sc_region_kernel,
        out_type=jax.ShapeDtypeStruct((2, _B, _K, 16), jnp.float32),
        mesh=plsc.VectorSubcoreMesh(core_axis_name="c", subcore_axis_name="s"),
        scratch_types=[
            pltpu.VMEM((48,), jnp.int32),
            pltpu.VMEM((48,), jnp.int32),
            pltpu.VMEM((_PADR, _W), jnp.float32),
            pltpu.VMEM((_PADR, _W), jnp.float32),
            pltpu.VMEM((16,), jnp.float32),
            pltpu.SemaphoreType.DMA,
            pltpu.SemaphoreType.DMA,
        ],
    )(ptbl, meta)

    totals = pl.pallas_call(
        _totals_kernel,
        in_specs=[
            pl.BlockSpec(memory_space=pltpu.SMEM),
            pl.BlockSpec(memory_space=pltpu.SMEM),
            pl.BlockSpec(memory_space=pl.ANY),
            pl.BlockSpec(memory_space=pl.ANY),
        ],
        out_specs=pl.BlockSpec(memory_space=pltpu.VMEM),
        out_shape=jax.ShapeDtypeStruct((_B, 128), jnp.float32),
        scratch_shapes=[
            pltpu.VMEM((_TNBUF, _TROWS, _W), jnp.float32),
            pltpu.VMEM((_TNBUF, _TROWS, _W), jnp.float32),
            pltpu.SemaphoreType.DMA((_TNBUF,)),
            pltpu.SemaphoreType.DMA((_TNBUF,)),
        ],
    )(sy, sx, ptbl, gtbl)

    out = pl.pallas_call(
        _combine_kernel,
        out_specs=pl.BlockSpec(memory_space=pltpu.SMEM),
        out_shape=jax.ShapeDtypeStruct((1, 1), jnp.float32),
    )(totals, sc_sums)
    return out[0, 0]
